# baseline (speedup 1.0000x reference)
import functools

import jax
import jax.numpy as jnp
from jax import lax
from jax.experimental import pallas as pl
from jax.experimental.pallas import tpu as pltpu
from jax.experimental.pallas import tpu_sc as plsc

_NC = 2
_NS = 16
_NW = _NC * _NS
_LANES = 16


@functools.partial(jax.jit, static_argnums=(3, 4, 5))
def _embed_lookup(idx_flat2d, tok_table, pos_table, B, T, D):
    n_tok = B * T
    b_per_w = n_tok // _NW
    n_seg = idx_flat2d.shape[1]
    seg_per_w = b_per_w // n_seg
    t_per_w = b_per_w
    mesh = plsc.VectorSubcoreMesh(core_axis_name="c", subcore_axis_name="s")

    @functools.partial(
        pl.kernel,
        out_type=jax.ShapeDtypeStruct((n_tok, D), jnp.float32),
        mesh=mesh,
        compiler_params=pltpu.CompilerParams(use_tc_tiling_on_sc=False),
        scratch_types=[
            pltpu.VMEM((seg_per_w, n_seg), jnp.int32),
            pltpu.VMEM((b_per_w, D), jnp.float32),
            pltpu.VMEM((b_per_w, D), jnp.float32),
            pltpu.SemaphoreType.DMA,
            pltpu.SemaphoreType.DMA,
        ],
    )
    def body(idx_hbm, tok_hbm, pos_hbm, out_hbm, idx_v, rows_v, pos_v, sem_g, sem_p):
        wid = lax.axis_index("s") * _NC + lax.axis_index("c")
        base = wid * b_per_w
        t0 = lax.rem(base, T)

        pltpu.sync_copy(idx_hbm.at[pl.ds(wid * seg_per_w, seg_per_w)], idx_v)
        pos_cp = pltpu.async_copy(pos_hbm.at[pl.ds(t0, t_per_w)], pos_v, sem_p)
        gathers = []
        for k in range(seg_per_w):
            gathers.append(pltpu.async_copy(
                tok_hbm.at[idx_v.at[k]],
                rows_v.at[pl.ds(k * n_seg, n_seg)],
                sem_g,
            ))
        pos_cp.wait()
        for g in gathers:
            g.wait()

        def row_add(i, carry):
            for j in range(D // _LANES):
                s = pl.ds(j * _LANES, _LANES)
                rows_v[i, s] = rows_v[i, s] + pos_v[i, s]
            return carry
        lax.fori_loop(0, b_per_w, row_add, 0)

        pltpu.sync_copy(rows_v, out_hbm.at[pl.ds(base, b_per_w)])

    return body(idx_flat2d, tok_table, pos_table)


def kernel(idx, tok_table, pos_table):
    B, T = idx.shape
    V, D = tok_table.shape
    idx_flat2d = idx.reshape(-1, 128).astype(jnp.int32)
    out = _embed_lookup(idx_flat2d, tok_table, pos_table, B, T, D)
    return out.reshape(B, T, D)


# speedup vs baseline: 1.6869x; 1.6869x over previous
import functools

import jax
import jax.numpy as jnp
from jax import lax
from jax.experimental import pallas as pl
from jax.experimental.pallas import tpu as pltpu
from jax.experimental.pallas import tpu_sc as plsc

_NC = 2
_NS = 16
_NW = _NC * _NS
_L = 16


@functools.partial(jax.jit, static_argnums=(3, 4, 5))
def _embed_lookup(idx_flat, tok_table, pos_table, B, T, D):
    n_tok = B * T
    b_per_w = n_tok // _NW
    n_chunk = b_per_w // _L
    mesh = plsc.VectorSubcoreMesh(core_axis_name="c", subcore_axis_name="s")

    @functools.partial(
        pl.kernel,
        out_type=jax.ShapeDtypeStruct((n_tok, D), jnp.float32),
        mesh=mesh,
        scratch_types=[
            pltpu.VMEM((b_per_w,), jnp.int32),
            pltpu.VMEM((b_per_w, D), jnp.float32),
            pltpu.VMEM((b_per_w, D), jnp.float32),
            pltpu.SemaphoreType.DMA,
            pltpu.SemaphoreType.DMA,
        ],
    )
    def body(idx_hbm, tok_hbm, pos_hbm, out_hbm,
             idx_v, rows_v, pos_v, sem_g, sem_p):
        wid = lax.axis_index("s") * _NC + lax.axis_index("c")
        base = wid * b_per_w
        t0 = lax.rem(base, T)

        pltpu.sync_copy(idx_hbm.at[pl.ds(base, b_per_w)], idx_v)
        pos_cp = pltpu.async_copy(pos_hbm.at[pl.ds(t0, b_per_w)], pos_v, sem_p)

        copies = []
        for ci in range(n_chunk):
            v = idx_v[pl.ds(ci * _L, _L)]
            for l in range(_L):
                i = ci * _L + l
                copies.append(pltpu.async_copy(
                    tok_hbm.at[pl.ds(v[l], 1)],
                    rows_v.at[pl.ds(i, 1)],
                    sem_g,
                ))
        pos_cp.wait()
        for cp in copies:
            cp.wait()

        def row_add(i, carry):
            for j in range(D // _L):
                s = pl.ds(j * _L, _L)
                rows_v[i, s] = rows_v[i, s] + pos_v[i, s]
            return carry
        lax.fori_loop(0, b_per_w, row_add, 0)

        pltpu.sync_copy(rows_v, out_hbm.at[pl.ds(base, b_per_w)])

    return body(idx_flat, tok_table, pos_table)


def kernel(idx, tok_table, pos_table):
    B, T = idx.shape
    V, D = tok_table.shape
    idx_flat = idx.reshape(-1).astype(jnp.int32)
    out = _embed_lookup(idx_flat, tok_table, pos_table, B, T, D)
    return out.reshape(B, T, D)


# speedup vs baseline: 2.3184x; 1.3744x over previous
import functools

import jax
import jax.numpy as jnp
from jax import lax
from jax.experimental import pallas as pl
from jax.experimental.pallas import tpu as pltpu
from jax.experimental.pallas import tpu_sc as plsc

_NC = 2
_NS = 16
_NW = _NC * _NS
_L = 16
_SEG = 32


@functools.partial(jax.jit, static_argnums=(3, 4, 5))
def _embed_lookup(idx_flat, tok_grouped, pos_table, B, T, D):
    n_tok = B * T
    b_per_w = n_tok // _NW
    n_seg = b_per_w // _SEG
    mesh = plsc.VectorSubcoreMesh(core_axis_name="c", subcore_axis_name="s")

    @functools.partial(
        pl.kernel,
        out_type=jax.ShapeDtypeStruct((n_tok, D), jnp.float32),
        mesh=mesh,
        scratch_types=[
            pltpu.VMEM((b_per_w,), jnp.int32),
            pltpu.VMEM((_SEG, 8, D), jnp.float32),
            pltpu.VMEM((b_per_w, D), jnp.float32),
            pltpu.VMEM((b_per_w, D), jnp.float32),
            pltpu.SemaphoreType.DMA,
            pltpu.SemaphoreType.DMA,
        ],
    )
    def body(idx_hbm, tok_hbm, pos_hbm, out_hbm,
             idx_v, groups_v, out_v, pos_v, sem_g, sem_p):
        wid = lax.axis_index("s") * _NC + lax.axis_index("c")
        base = wid * b_per_w
        t0 = lax.rem(base, T)

        pltpu.sync_copy(idx_hbm.at[pl.ds(base, b_per_w)], idx_v)
        pos_cp = pltpu.async_copy(pos_hbm.at[pl.ds(t0, b_per_w)], pos_v, sem_p)
        pos_cp.wait()

        for sgi in range(n_seg):
            copies = []
            for ci in range(_SEG // _L):
                v = lax.shift_right_logical(
                    idx_v[pl.ds(sgi * _SEG + ci * _L, _L)], 3)
                for l in range(_L):
                    copies.append(pltpu.async_copy(
                        tok_hbm.at[pl.ds(v[l], 1)],
                        groups_v.at[pl.ds(ci * _L + l, 1)],
                        sem_g,
                    ))
            for cp in copies:
                cp.wait()

            def seg_body(ci, carry, sgi=sgi):
                row0 = sgi * _SEG + ci * _L
                sub = idx_v[pl.ds(row0, _L)] & 7
                for l in range(_L):
                    r = sub[l]
                    i = row0 + l
                    for j in range(D // _L):
                        s = pl.ds(j * _L, _L)
                        out_v[i, s] = groups_v[ci * _L + l, r, s] + pos_v[i, s]
                return carry
            lax.fori_loop(0, _SEG // _L, seg_body, 0)

        pltpu.sync_copy(out_v, out_hbm.at[pl.ds(base, b_per_w)])

    return body(idx_flat, tok_grouped, pos_table)


def kernel(idx, tok_table, pos_table):
    B, T = idx.shape
    V, D = tok_table.shape
    idx_flat = idx.reshape(-1).astype(jnp.int32)
    tok_grouped = tok_table.reshape(V // 8, 8, D)
    out = _embed_lookup(idx_flat, tok_grouped, pos_table, B, T, D)
    return out.reshape(B, T, D)
